# Initial kernel scaffold; baseline (speedup 1.0000x reference)
import jax
import jax.numpy as jnp
from jax.experimental import pallas as pl


def kernel(x, edge_index, edge_attr, W_e0, b_e0, W1_0, b1_0, W2_0, b2_0, W_e1, b_e1, W1_1, b1_1, W2_1, b2_1, W_e2, b_e2, W1_2, b1_2, W2_2, b2_2):
    raise NotImplementedError("write your pallas kernel here")


# speedup vs baseline: 2.2439x; 2.2439x over previous
import functools

import jax
import jax.numpy as jnp
from jax import lax
from jax.experimental import pallas as pl
from jax.experimental.pallas import tpu as pltpu
from jax.experimental.pallas import tpu_sc as plsc

N_NODES = 10000
N_EDGES = 320000
D = 128

NC = 2
NS = 16
NW = NC * NS
E_PER_W = N_EDGES // NW
CHUNK = 80
N_CHUNK = E_PER_W // CHUNK
K = 5
SUPER = CHUNK * K
N_SUPER = E_PER_W // SUPER
HALF = 5120
ACC_ROWS = HALF + 128
ZERO_PER_TILE = ACC_ROWS // NS
DUMP_PER_TILE = HALF // NS
E_PER_TILE = N_EDGES // NS
N_CHUNK_SC = E_PER_TILE // CHUNK
N_SUPER_SC = E_PER_TILE // SUPER
OUT_ROWS = 2 * HALF

_mesh = plsc.VectorSubcoreMesh(core_axis_name="c", subcore_axis_name="s")


@functools.partial(
    pl.kernel, mesh=_mesh,
    out_type=jax.ShapeDtypeStruct((N_EDGES, D), jnp.float32),
    scratch_types=[
        pltpu.VMEM((N_CHUNK, CHUNK), jnp.int32),
        pltpu.VMEM((SUPER, D), jnp.float32),
        pltpu.SemaphoreType.DMA,
    ],
)
def _sc_gather(table_hbm, idx_hbm, out_hbm, idx_v, rows_v, sem):
    wid = lax.axis_index("s") * NC + lax.axis_index("c")
    pltpu.sync_copy(idx_hbm.at[wid], idx_v)

    @pl.loop(0, N_SUPER)
    def _(i):
        cps = [
            pltpu.async_copy(
                table_hbm.at[idx_v.at[i * K + t]],
                rows_v.at[pl.ds(t * CHUNK, CHUNK)],
                sem,
            )
            for t in range(K)
        ]
        for cp in cps:
            cp.wait()
        pltpu.sync_copy(
            rows_v, out_hbm.at[pl.ds(wid * E_PER_W + i * SUPER, SUPER)]
        )


@functools.partial(
    pl.kernel, mesh=_mesh,
    out_type=jax.ShapeDtypeStruct((OUT_ROWS, D), jnp.float32),
    scratch_types=[
        pltpu.VMEM((N_CHUNK_SC, CHUNK), jnp.int32),
        pltpu.VMEM((CHUNK,), jnp.int32),
        pltpu.VMEM((SUPER, D), jnp.float32),
        pltpu.VMEM_SHARED((ACC_ROWS, D), jnp.float32),
    ],
)
def _sc_scatter_add(msg_hbm, idx_hbm, zeros_hbm, out_hbm,
                    idx_v, idx2_v, upd_v, accum):
    cid = lax.axis_index("c")
    sid = lax.axis_index("s")
    base = cid * HALF
    pltpu.sync_copy(idx_hbm.at[sid], idx_v)
    pltpu.sync_copy(zeros_hbm, accum.at[pl.ds(sid * ZERO_PER_TILE,
                                              ZERO_PER_TILE)])
    plsc.subcore_barrier()

    @pl.loop(0, N_SUPER_SC)
    def _(i):
        pltpu.sync_copy(
            msg_hbm.at[pl.ds(sid * E_PER_TILE + i * SUPER, SUPER)], upd_v
        )
        for t in range(K):
            for q in range(CHUNK // 16):
                v = idx_v[i * K + t, pl.ds(q * 16, 16)] - base
                inb = (v >= 0) & (v < HALF)
                dummy = jnp.full((16,), HALF + sid * 8 + q, jnp.int32)
                idx2_v[pl.ds(q * 16, 16)] = jnp.where(inb, v, dummy)
            pltpu.sync_copy(
                upd_v.at[pl.ds(t * CHUNK, CHUNK)],
                accum.at[idx2_v],
                add=True,
            )

    plsc.subcore_barrier()
    pltpu.sync_copy(
        accum.at[pl.ds(sid * DUMP_PER_TILE, DUMP_PER_TILE)],
        out_hbm.at[pl.ds(base + sid * DUMP_PER_TILE, DUMP_PER_TILE)],
    )


def _msg_body(g_ref, ea_ref, w_ref, b_ref, out_ref):
    e = jnp.dot(ea_ref[...], w_ref[...],
                preferred_element_type=jnp.float32,
                precision=lax.Precision.HIGHEST)
    out_ref[...] = jnp.maximum(g_ref[...] + e + b_ref[...], 0.0)


def _tc_msg(g, ea, w_et, b_e):
    blk = 2000
    grid = N_EDGES // blk
    return pl.pallas_call(
        _msg_body,
        grid=(grid,),
        in_specs=[
            pl.BlockSpec((blk, D), lambda i: (i, 0)),
            pl.BlockSpec((blk, 16), lambda i: (i, 0)),
            pl.BlockSpec((16, D), lambda i: (0, 0)),
            pl.BlockSpec((1, D), lambda i: (0, 0)),
        ],
        out_specs=pl.BlockSpec((blk, D), lambda i: (i, 0)),
        out_shape=jax.ShapeDtypeStruct((N_EDGES, D), jnp.float32),
    )(g, ea, w_et, b_e)


def _node_body(h_ref, p_ref, w1_ref, b1_ref, w2_ref, b2_ref,
               out_ref, *, relu_out):
    z = h_ref[...] + p_ref[...]
    t = jnp.maximum(
        jnp.dot(z, w1_ref[...], preferred_element_type=jnp.float32,
                precision=lax.Precision.HIGHEST) + b1_ref[...], 0.0)
    o = jnp.dot(t, w2_ref[...], preferred_element_type=jnp.float32,
                precision=lax.Precision.HIGHEST) + b2_ref[...]
    if relu_out:
        o = jnp.maximum(o, 0.0)
    out_ref[...] = o


def _tc_node(h, parts, w1t, b1, w2t, b2, relu_out):
    blk = 1000
    grid = N_NODES // blk
    return pl.pallas_call(
        functools.partial(_node_body, relu_out=relu_out),
        grid=(grid,),
        in_specs=[
            pl.BlockSpec((blk, D), lambda i: (i, 0)),
            pl.BlockSpec((blk, D), lambda i: (i, 0)),
            pl.BlockSpec((D, D), lambda i: (0, 0)),
            pl.BlockSpec((1, D), lambda i: (0, 0)),
            pl.BlockSpec((D, D), lambda i: (0, 0)),
            pl.BlockSpec((1, D), lambda i: (0, 0)),
        ],
        out_specs=pl.BlockSpec((blk, D), lambda i: (i, 0)),
        out_shape=jax.ShapeDtypeStruct((N_NODES, D), jnp.float32),
    )(h, parts, w1t, b1, w2t, b2)


def _node_mean_body(h_ref, p_ref, w1_ref, b1_ref, w2_ref, b2_ref,
                    out_ref):
    i = pl.program_id(0)
    z = h_ref[...] + p_ref[...]
    t = jnp.maximum(
        jnp.dot(z, w1_ref[...], preferred_element_type=jnp.float32,
                precision=lax.Precision.HIGHEST) + b1_ref[...], 0.0)
    o = jnp.dot(t, w2_ref[...], preferred_element_type=jnp.float32,
                precision=lax.Precision.HIGHEST) + b2_ref[...]

    @pl.when(i == 0)
    def _():
        out_ref[...] = jnp.zeros_like(out_ref)

    out_ref[...] += jnp.sum(o, axis=0, keepdims=True) * (1.0 / N_NODES)


def _tc_node_mean(h, parts, w1t, b1, w2t, b2):
    blk = 1000
    grid = N_NODES // blk
    return pl.pallas_call(
        _node_mean_body,
        grid=(grid,),
        in_specs=[
            pl.BlockSpec((blk, D), lambda i: (i, 0)),
            pl.BlockSpec((blk, D), lambda i: (i, 0)),
            pl.BlockSpec((D, D), lambda i: (0, 0)),
            pl.BlockSpec((1, D), lambda i: (0, 0)),
            pl.BlockSpec((D, D), lambda i: (0, 0)),
            pl.BlockSpec((1, D), lambda i: (0, 0)),
        ],
        out_specs=pl.BlockSpec((1, D), lambda i: (0, 0)),
        out_shape=jax.ShapeDtypeStruct((1, D), jnp.float32),
    )(h, parts, w1t, b1, w2t, b2)


def kernel(x, edge_index, edge_attr,
           W_e0, b_e0, W1_0, b1_0, W2_0, b2_0,
           W_e1, b_e1, W1_1, b1_1, W2_1, b2_1,
           W_e2, b_e2, W1_2, b1_2, W2_2, b2_2):
    src3 = jnp.asarray(edge_index[0], jnp.int32).reshape(NW, N_CHUNK, CHUNK)
    dst3 = jnp.asarray(edge_index[1], jnp.int32).reshape(NS, N_CHUNK_SC, CHUNK)
    zeros = jnp.zeros((ZERO_PER_TILE, D), jnp.float32)

    params = [
        (W_e0, b_e0, W1_0, b1_0, W2_0, b2_0),
        (W_e1, b_e1, W1_1, b1_1, W2_1, b2_1),
        (W_e2, b_e2, W1_2, b1_2, W2_2, b2_2),
    ]
    h = x
    for l, (w_e, b_e, w1, b1, w2, b2) in enumerate(params):
        g = _sc_gather(h, src3)
        msg = _tc_msg(g, edge_attr, w_e.T, b_e.reshape(1, D))
        parts = _sc_scatter_add(msg, dst3, zeros)
        if l < 2:
            h = _tc_node(h, parts, w1.T, b1.reshape(1, D),
                         w2.T, b2.reshape(1, D), relu_out=True)
        else:
            h = _tc_node_mean(h, parts, w1.T, b1.reshape(1, D),
                              w2.T, b2.reshape(1, D))
    return h


# speedup vs baseline: 2.5881x; 1.1534x over previous
import functools

import jax
import jax.numpy as jnp
from jax import lax
from jax.experimental import pallas as pl
from jax.experimental.pallas import tpu as pltpu
from jax.experimental.pallas import tpu_sc as plsc

N_NODES = 10000
N_EDGES = 320000
D = 128

NC = 2
NS = 16
NW = NC * NS
E_PER_W = N_EDGES // NW
CHUNK = 80
N_CHUNK = E_PER_W // CHUNK
K = 5
SUPER = CHUNK * K
N_SUPER = E_PER_W // SUPER
HALF = 5120
ACC_ROWS = HALF + 128
ZERO_PER_TILE = ACC_ROWS // NS
DUMP_PER_TILE = HALF // NS
E_PER_TILE = N_EDGES // NS
N_CHUNK_SC = E_PER_TILE // CHUNK
K_SC = 2
SUPER_SC = CHUNK * K_SC
N_SUPER_SC = E_PER_TILE // SUPER_SC
OUT_ROWS = 2 * HALF

_mesh = plsc.VectorSubcoreMesh(core_axis_name="c", subcore_axis_name="s")


def _pipeline2(n, stage_in, wait_in, stage_out, wait_out):
    def body(i, b):
        wait_in(b)
        stage_out(i, b)

    stage_in(0, 0)
    stage_in(1, 1)
    n_even = n - (n % 2)

    @pl.loop(0, max(n_even - 2, 0), step=2)
    def _(i):
        body(i, 0)
        wait_out(0)
        stage_in(i + 2, 0)
        body(i + 1, 1)
        wait_out(1)
        stage_in(i + 3, 1)

    if n % 2:
        body(n - 3, 0)
        wait_out(0)
        stage_in(n - 1, 0)
        body(n - 2, 1)
        body(n - 1, 0)
    else:
        body(n - 2, 0)
        body(n - 1, 1)
    wait_out(0)
    wait_out(1)


@functools.partial(
    pl.kernel, mesh=_mesh,
    out_type=jax.ShapeDtypeStruct((N_EDGES, D), jnp.float32),
    scratch_types=[
        pltpu.VMEM((N_CHUNK, CHUNK), jnp.int32),
        pltpu.VMEM((2, SUPER, D), jnp.float32),
        pltpu.SemaphoreType.DMA,
        pltpu.SemaphoreType.DMA,
        pltpu.SemaphoreType.DMA,
        pltpu.SemaphoreType.DMA,
    ],
)
def _sc_gather(table_hbm, idx_hbm, out_hbm, idx_v, rows_v, sg0, sg1, so0, so1):
    wid = lax.axis_index("s") * NC + lax.axis_index("c")
    base = wid * E_PER_W
    pltpu.sync_copy(idx_hbm.at[wid], idx_v)
    sg = (sg0, sg1)
    so = (so0, so1)

    def stage_in(i, b):
        for t in range(K):
            pltpu.async_copy(
                table_hbm.at[idx_v.at[i * K + t]],
                rows_v.at[b].at[pl.ds(t * CHUNK, CHUNK)],
                sg[b],
            )

    def wait_in(b):
        for _ in range(K):
            pltpu.make_async_copy(
                table_hbm.at[idx_v.at[0]],
                rows_v.at[b].at[pl.ds(0, CHUNK)],
                sg[b],
            ).wait()

    def stage_out(i, b):
        pltpu.async_copy(
            rows_v.at[b], out_hbm.at[pl.ds(base + i * SUPER, SUPER)], so[b],
        )

    def wait_out(b):
        pltpu.make_async_copy(
            rows_v.at[b], out_hbm.at[pl.ds(0, SUPER)], so[b],
        ).wait()

    _pipeline2(N_SUPER, stage_in, wait_in, stage_out, wait_out)


@functools.partial(
    pl.kernel, mesh=_mesh,
    out_type=jax.ShapeDtypeStruct((OUT_ROWS, D), jnp.float32),
    scratch_types=[
        pltpu.VMEM((N_CHUNK_SC, CHUNK), jnp.int32),
        pltpu.VMEM((2, SUPER_SC, D), jnp.float32),
        pltpu.SemaphoreType.DMA,
        pltpu.SemaphoreType.DMA,
        pltpu.SemaphoreType.DMA,
        pltpu.SemaphoreType.DMA,
        pltpu.VMEM_SHARED((ACC_ROWS, D), jnp.float32),
    ],
)
def _sc_scatter_add(msg_hbm, idx_hbm, zeros_hbm, out_hbm,
                    idx_v, upd_v, sm0, sm1, ss0, ss1, accum):
    cid = lax.axis_index("c")
    sid = lax.axis_index("s")
    base = cid * HALF
    pltpu.sync_copy(idx_hbm.at[sid], idx_v)
    pltpu.sync_copy(zeros_hbm, accum.at[pl.ds(sid * ZERO_PER_TILE,
                                              ZERO_PER_TILE)])

    @pl.loop(0, N_CHUNK_SC)
    def _(j):
        for q in range(CHUNK // 16):
            v = idx_v[j, pl.ds(q * 16, 16)] - base
            inb = (v >= 0) & (v < HALF)
            dummy = jnp.full((16,), HALF + sid * 8 + q, jnp.int32)
            idx_v[j, pl.ds(q * 16, 16)] = jnp.where(inb, v, dummy)

    plsc.subcore_barrier()
    sm = (sm0, sm1)
    ss = (ss0, ss1)

    def stage_in(i, b):
        pltpu.async_copy(
            msg_hbm.at[pl.ds(sid * E_PER_TILE + i * SUPER_SC, SUPER_SC)],
            upd_v.at[b], sm[b],
        )

    def wait_in(b):
        pltpu.make_async_copy(
            msg_hbm.at[pl.ds(0, SUPER_SC)], upd_v.at[b], sm[b],
        ).wait()

    def stage_out(i, b):
        for t in range(K_SC):
            pltpu.async_copy(
                upd_v.at[b].at[pl.ds(t * CHUNK, CHUNK)],
                accum.at[idx_v.at[i * K_SC + t]],
                ss[b], add=True,
            )

    def wait_out(b):
        for _ in range(K_SC):
            pltpu.make_async_copy(
                upd_v.at[b].at[pl.ds(0, CHUNK)],
                accum.at[idx_v.at[0]],
                ss[b],
            ).wait()

    _pipeline2(N_SUPER_SC, stage_in, wait_in, stage_out, wait_out)

    plsc.subcore_barrier()
    pltpu.sync_copy(
        accum.at[pl.ds(sid * DUMP_PER_TILE, DUMP_PER_TILE)],
        out_hbm.at[pl.ds(base + sid * DUMP_PER_TILE, DUMP_PER_TILE)],
    )


def _msg_body(g_ref, ea_ref, w_ref, b_ref, out_ref):
    e = jnp.dot(ea_ref[...], w_ref[...],
                preferred_element_type=jnp.float32,
                precision=lax.Precision.HIGHEST)
    out_ref[...] = jnp.maximum(g_ref[...] + e + b_ref[...], 0.0)


def _tc_msg(g, ea, w_et, b_e):
    blk = 2000
    return pl.pallas_call(
        _msg_body,
        grid=(N_EDGES // blk,),
        in_specs=[
            pl.BlockSpec((blk, D), lambda i: (i, 0)),
            pl.BlockSpec((blk, 16), lambda i: (i, 0)),
            pl.BlockSpec((16, D), lambda i: (0, 0)),
            pl.BlockSpec((1, D), lambda i: (0, 0)),
        ],
        out_specs=pl.BlockSpec((blk, D), lambda i: (i, 0)),
        out_shape=jax.ShapeDtypeStruct((N_EDGES, D), jnp.float32),
    )(g, ea, w_et, b_e)


def _node_body(h_ref, p_ref, w1_ref, b1_ref, w2_ref, b2_ref,
               out_ref, *, relu_out):
    z = h_ref[...] + p_ref[...]
    t = jnp.maximum(
        jnp.dot(z, w1_ref[...], preferred_element_type=jnp.float32,
                precision=lax.Precision.HIGHEST) + b1_ref[...], 0.0)
    o = jnp.dot(t, w2_ref[...], preferred_element_type=jnp.float32,
                precision=lax.Precision.HIGHEST) + b2_ref[...]
    if relu_out:
        o = jnp.maximum(o, 0.0)
    out_ref[...] = o


_NODE_SPECS = [
    pl.BlockSpec((1000, D), lambda i: (i, 0)),
    pl.BlockSpec((1000, D), lambda i: (i, 0)),
    pl.BlockSpec((D, D), lambda i: (0, 0)),
    pl.BlockSpec((1, D), lambda i: (0, 0)),
    pl.BlockSpec((D, D), lambda i: (0, 0)),
    pl.BlockSpec((1, D), lambda i: (0, 0)),
]


def _tc_node(h, p, w1t, b1, w2t, b2, relu_out):
    return pl.pallas_call(
        functools.partial(_node_body, relu_out=relu_out),
        grid=(N_NODES // 1000,),
        in_specs=_NODE_SPECS,
        out_specs=pl.BlockSpec((1000, D), lambda i: (i, 0)),
        out_shape=jax.ShapeDtypeStruct((N_NODES, D), jnp.float32),
    )(h, p, w1t, b1, w2t, b2)


def _node_mean_body(h_ref, p_ref, w1_ref, b1_ref, w2_ref, b2_ref, out_ref):
    i = pl.program_id(0)
    z = h_ref[...] + p_ref[...]
    t = jnp.maximum(
        jnp.dot(z, w1_ref[...], preferred_element_type=jnp.float32,
                precision=lax.Precision.HIGHEST) + b1_ref[...], 0.0)
    o = jnp.dot(t, w2_ref[...], preferred_element_type=jnp.float32,
                precision=lax.Precision.HIGHEST) + b2_ref[...]

    @pl.when(i == 0)
    def _():
        out_ref[...] = jnp.zeros_like(out_ref)

    out_ref[...] += jnp.sum(o, axis=0, keepdims=True) * (1.0 / N_NODES)


def _tc_node_mean(h, p, w1t, b1, w2t, b2):
    return pl.pallas_call(
        _node_mean_body,
        grid=(N_NODES // 1000,),
        in_specs=_NODE_SPECS,
        out_specs=pl.BlockSpec((1, D), lambda i: (0, 0)),
        out_shape=jax.ShapeDtypeStruct((1, D), jnp.float32),
    )(h, p, w1t, b1, w2t, b2)


def kernel(x, edge_index, edge_attr,
           W_e0, b_e0, W1_0, b1_0, W2_0, b2_0,
           W_e1, b_e1, W1_1, b1_1, W2_1, b2_1,
           W_e2, b_e2, W1_2, b1_2, W2_2, b2_2):
    src3 = jnp.asarray(edge_index[0], jnp.int32).reshape(NW, N_CHUNK, CHUNK)
    dst3 = jnp.asarray(edge_index[1], jnp.int32).reshape(NS, N_CHUNK_SC, CHUNK)
    zeros = jnp.zeros((ZERO_PER_TILE, D), jnp.float32)

    params = [
        (W_e0, b_e0, W1_0, b1_0, W2_0, b2_0),
        (W_e1, b_e1, W1_1, b1_1, W2_1, b2_1),
        (W_e2, b_e2, W1_2, b1_2, W2_2, b2_2),
    ]
    h = x
    for l, (w_e, b_e, w1, b1, w2, b2) in enumerate(params):
        g = _sc_gather(h, src3)
        msg = _tc_msg(g, edge_attr, w_e.T, b_e.reshape(1, D))
        p = _sc_scatter_add(msg, dst3, zeros)
        if l < 2:
            h = _tc_node(h, p, w1.T, b1.reshape(1, D),
                         w2.T, b2.reshape(1, D), relu_out=True)
        else:
            h = _tc_node_mean(h, p, w1.T, b1.reshape(1, D),
                              w2.T, b2.reshape(1, D))
    return h
